# baseline (speedup 1.0000x reference)
import jax
import jax.numpy as jnp
import numpy as np
from jax.experimental import pallas as pl
from jax.experimental.pallas import tpu as pltpu

_B, _H, _S, _D = 1, 12, 2048, 64
_BLK = 64
_SEL = 16
_NB = _S // _BLK
_TQ = 512
_TK = 512
_BPT = _TK // _BLK
_DV = _D + 1
_NEG = -1e30


def _nsa_tc_kernel(bi_ref, q_ref, m_ref, k_ref, v_ref, o_ref):
    qt = pl.program_id(1)
    q = q_ref[0] * jnp.bfloat16(0.125)
    bi = bi_ref[0]
    mm = m_ref[0]
    e8 = (jax.lax.broadcasted_iota(jnp.int32, (_BPT, _TK), 1) // _BLK ==
          jax.lax.broadcasted_iota(jnp.int32, (_BPT, _TK), 0)
          ).astype(jnp.bfloat16)

    qpos = qt * _TQ + jax.lax.broadcasted_iota(jnp.int32, (_TQ, 1), 0)
    t = jnp.left_shift(jnp.int32(1), bi)
    w = _SEL
    while w > 1:
        w //= 2
        t = t[:, :w] | t[:, w:2 * w]
    bits = t | jnp.left_shift(jnp.int32(1), qpos // _BLK)

    b8 = jax.lax.broadcasted_iota(jnp.int32, (_TQ, _BPT), 1)

    def masked_scores(i):
        k = k_ref[0, pl.ds(i * _TK, _TK), :]
        s = jax.lax.dot_general(
            q, k, (((1,), (1,)), ((), ())),
            preferred_element_type=jnp.float32)
        win = jax.lax.shift_right_logical(bits, i * _BPT)
        ok = (jax.lax.shift_right_logical(win, b8) & 1) != 0
        bias8 = jnp.where(ok, jnp.float32(0), jnp.float32(_NEG)
                          ).astype(jnp.bfloat16)
        return s + jax.lax.dot_general(
            bias8, e8, (((1,), (0,)), ((), ())),
            preferred_element_type=jnp.float32)

    def pv(e, i):
        v = v_ref[0, pl.ds(i * _TK, _TK), :]
        return jax.lax.dot_general(
            e.astype(jnp.bfloat16), v, (((1,), (0,)), ((), ())),
            preferred_element_type=jnp.float32)

    r_io = jax.lax.broadcasted_iota(jnp.int32, (_TQ, _TK), 0)
    c_io = jax.lax.broadcasted_iota(jnp.int32, (_TQ, _TK), 1)
    s = jnp.where(r_io >= c_io, masked_scores(qt), jnp.float32(_NEG))
    acc = pv(jnp.exp(s - mm), qt)

    def body(i, acc):
        return acc + pv(jnp.exp(masked_scores(i) - mm), i)

    acc = jax.lax.fori_loop(0, qt, body, acc)

    o_ref[0] = acc[:, :_D] / acc[:, _D:]


def kernel(Q, K, V, BlockIndices):
    B, H, S, D = Q.shape
    q = Q.reshape(H, S, D).astype(jnp.bfloat16)
    k = K.reshape(H, S, D).astype(jnp.bfloat16)
    bi = BlockIndices.reshape(H, S, _SEL).astype(jnp.int32)
    v = V.reshape(H, S, D).astype(jnp.bfloat16)
    v = jnp.concatenate([v, jnp.ones((H, S, 1), jnp.bfloat16)], axis=2)
    qf = q.astype(jnp.float32)
    kf = k.astype(jnp.float32)
    qn = jnp.sqrt(jnp.sum(qf * qf, axis=2, keepdims=True))
    kn = jnp.max(jnp.sqrt(jnp.sum(kf * kf, axis=2)), axis=1)
    mb = 0.125 * qn * kn[:, None, None] * 1.001 + 1e-6

    grid = (H, S // _TQ)
    out = pl.pallas_call(
        _nsa_tc_kernel,
        grid=grid,
        in_specs=[
            pl.BlockSpec((1, _TQ, _SEL), lambda h, t: (h, t, 0)),
            pl.BlockSpec((1, _TQ, D), lambda h, t: (h, t, 0)),
            pl.BlockSpec((1, _TQ, 1), lambda h, t: (h, t, 0)),
            pl.BlockSpec((1, S, D), lambda h, t: (h, 0, 0)),
            pl.BlockSpec((1, S, _DV), lambda h, t: (h, 0, 0)),
        ],
        out_specs=pl.BlockSpec((1, _TQ, D), lambda h, t: (h, t, 0)),
        out_shape=jax.ShapeDtypeStruct((H, S, D), jnp.float32),
    )(bi, q, mb, k, v)
    return out.reshape(B, H, S, D)


# speedup vs baseline: 1.0302x; 1.0302x over previous
import jax
import jax.numpy as jnp
import numpy as np
from jax.experimental import pallas as pl
from jax.experimental.pallas import tpu as pltpu

_B, _H, _S, _D = 1, 12, 2048, 64
_BLK = 64
_SEL = 16
_NB = _S // _BLK
_TQ = 512
_TK = 512
_BPT = _TK // _BLK
_NEG = -1e30


def _nsa_tc_kernel(bi_ref, q_ref, m_ref, k_ref, v_ref, o_ref):
    qt = pl.program_id(1)
    q = q_ref[0] * jnp.bfloat16(0.125)
    bi = bi_ref[0]
    mm = m_ref[0]
    e8 = (jax.lax.broadcasted_iota(jnp.int32, (_BPT, _TK), 1) // _BLK ==
          jax.lax.broadcasted_iota(jnp.int32, (_BPT, _TK), 0)
          ).astype(jnp.bfloat16)

    qpos = qt * _TQ + jax.lax.broadcasted_iota(jnp.int32, (_TQ, 1), 0)
    t = jnp.left_shift(jnp.int32(1), bi)
    w = _SEL
    while w > 1:
        w //= 2
        t = t[:, :w] | t[:, w:2 * w]
    bits = t | jnp.left_shift(jnp.int32(1), qpos // _BLK)

    b8 = jax.lax.broadcasted_iota(jnp.int32, (_TQ, _BPT), 1)

    def masked_scores(i):
        k = k_ref[0, pl.ds(i * _TK, _TK), :]
        s = jax.lax.dot_general(
            q, k, (((1,), (1,)), ((), ())),
            preferred_element_type=jnp.float32)
        win = jax.lax.shift_right_logical(bits, i * _BPT)
        ok = (jax.lax.shift_right_logical(win, b8) & 1) != 0
        bias8 = jnp.where(ok, jnp.float32(0), jnp.float32(_NEG)
                          ).astype(jnp.bfloat16)
        return s + jax.lax.dot_general(
            bias8, e8, (((1,), (0,)), ((), ())),
            preferred_element_type=jnp.float32)

    def pv(e, i):
        v = v_ref[0, pl.ds(i * _TK, _TK), :]
        return jax.lax.dot_general(
            e.astype(jnp.bfloat16), v, (((1,), (0,)), ((), ())),
            preferred_element_type=jnp.float32)

    r_io = jax.lax.broadcasted_iota(jnp.int32, (_TQ, _TK), 0)
    c_io = jax.lax.broadcasted_iota(jnp.int32, (_TQ, _TK), 1)
    s = jnp.where(r_io >= c_io, masked_scores(qt), jnp.float32(_NEG))
    e = jnp.exp(s - mm)
    acc = pv(e, qt)
    l = jnp.sum(e, axis=-1, keepdims=True)

    def body(i, carry):
        l, acc = carry
        e = jnp.exp(masked_scores(i) - mm)
        return l + jnp.sum(e, axis=-1, keepdims=True), acc + pv(e, i)

    l, acc = jax.lax.fori_loop(0, qt, body, (l, acc))

    o_ref[0] = acc / l


def kernel(Q, K, V, BlockIndices):
    B, H, S, D = Q.shape
    q = Q.reshape(H, S, D).astype(jnp.bfloat16)
    k = K.reshape(H, S, D).astype(jnp.bfloat16)
    bi = BlockIndices.reshape(H, S, _SEL).astype(jnp.int32)
    v = V.reshape(H, S, D).astype(jnp.bfloat16)
    qf = q.astype(jnp.float32)
    kf = k.astype(jnp.float32)
    qn = jnp.sqrt(jnp.sum(qf * qf, axis=2, keepdims=True))
    kn = jnp.max(jnp.sqrt(jnp.sum(kf * kf, axis=2)), axis=1)
    mb = 0.125 * qn * kn[:, None, None] * 1.001 + 1e-6

    grid = (H, S // _TQ)
    out = pl.pallas_call(
        _nsa_tc_kernel,
        grid=grid,
        in_specs=[
            pl.BlockSpec((1, _TQ, _SEL), lambda h, t: (h, t, 0)),
            pl.BlockSpec((1, _TQ, D), lambda h, t: (h, t, 0)),
            pl.BlockSpec((1, _TQ, 1), lambda h, t: (h, t, 0)),
            pl.BlockSpec((1, S, D), lambda h, t: (h, 0, 0)),
            pl.BlockSpec((1, S, D), lambda h, t: (h, 0, 0)),
        ],
        out_specs=pl.BlockSpec((1, _TQ, D), lambda h, t: (h, t, 0)),
        out_shape=jax.ShapeDtypeStruct((H, S, D), jnp.float32),
    )(bi, q, mb, k, v)
    return out.reshape(B, H, S, D)


# speedup vs baseline: 1.0672x; 1.0360x over previous
import jax
import jax.numpy as jnp
import numpy as np
from jax.experimental import pallas as pl
from jax.experimental.pallas import tpu as pltpu

_B, _H, _S, _D = 1, 12, 2048, 64
_BLK = 64
_SEL = 16
_NB = _S // _BLK
_TQ = 512
_TK = 512
_BPT = _TK // _BLK
_NEG = -1e30


def _nsa_tc_kernel(bi_ref, q_ref, k_ref, v_ref, o_ref):
    qt = pl.program_id(1)
    q = q_ref[0] * jnp.bfloat16(0.125)
    bi = bi_ref[0]
    e8 = (jax.lax.broadcasted_iota(jnp.int32, (_BPT, _TK), 1) // _BLK ==
          jax.lax.broadcasted_iota(jnp.int32, (_BPT, _TK), 0)
          ).astype(jnp.bfloat16)

    qpos = qt * _TQ + jax.lax.broadcasted_iota(jnp.int32, (_TQ, 1), 0)
    t = jnp.left_shift(jnp.int32(1), bi)
    w = _SEL
    while w > 1:
        w //= 2
        t = t[:, :w] | t[:, w:2 * w]
    bits = t | jnp.left_shift(jnp.int32(1), qpos // _BLK)

    b8 = jax.lax.broadcasted_iota(jnp.int32, (_TQ, _BPT), 1)

    def masked_scores(i):
        k = k_ref[0, pl.ds(i * _TK, _TK), :]
        s = jax.lax.dot_general(
            q, k, (((1,), (1,)), ((), ())),
            preferred_element_type=jnp.float32)
        win = jax.lax.shift_right_logical(bits, i * _BPT)
        ok = (jax.lax.shift_right_logical(win, b8) & 1) != 0
        bias8 = jnp.where(ok, jnp.float32(0), jnp.float32(_NEG)
                          ).astype(jnp.bfloat16)
        return s + jax.lax.dot_general(
            bias8, e8, (((1,), (0,)), ((), ())),
            preferred_element_type=jnp.float32)

    def pv(e, i):
        v = v_ref[0, pl.ds(i * _TK, _TK), :]
        return jax.lax.dot_general(
            e.astype(jnp.bfloat16), v, (((1,), (0,)), ((), ())),
            preferred_element_type=jnp.float32)

    r_io = jax.lax.broadcasted_iota(jnp.int32, (_TQ, _TK), 0)
    c_io = jax.lax.broadcasted_iota(jnp.int32, (_TQ, _TK), 1)
    s = jnp.where(r_io >= c_io, masked_scores(qt), jnp.float32(_NEG))
    m = jnp.max(s, axis=-1, keepdims=True)
    e = jnp.exp(s - m)
    l = jnp.sum(e, axis=-1, keepdims=True)
    acc = pv(e, qt)

    def body(i, carry):
        m, l, acc = carry
        s = masked_scores(i)
        m_new = jnp.maximum(m, jnp.max(s, axis=-1, keepdims=True))
        alpha = jnp.exp(m - m_new)
        e = jnp.exp(s - m_new)
        acc = acc * alpha + pv(e, i)
        l = l * alpha + jnp.sum(e, axis=-1, keepdims=True)
        return m_new, l, acc

    _, l, acc = jax.lax.fori_loop(0, qt, body, (m, l, acc))

    o_ref[0] = acc / l


def kernel(Q, K, V, BlockIndices):
    B, H, S, D = Q.shape
    q = Q.reshape(H, S, D).astype(jnp.bfloat16)
    k = K.reshape(H, S, D).astype(jnp.bfloat16)
    v = V.reshape(H, S, D).astype(jnp.bfloat16)
    bi = BlockIndices.reshape(H, S, _SEL).astype(jnp.int32)

    grid = (H, S // _TQ)
    out = pl.pallas_call(
        _nsa_tc_kernel,
        grid=grid,
        in_specs=[
            pl.BlockSpec((1, _TQ, _SEL), lambda h, t: (h, t, 0)),
            pl.BlockSpec((1, _TQ, D), lambda h, t: (h, t, 0)),
            pl.BlockSpec((1, S, D), lambda h, t: (h, 0, 0)),
            pl.BlockSpec((1, S, D), lambda h, t: (h, 0, 0)),
        ],
        out_specs=pl.BlockSpec((1, _TQ, D), lambda h, t: (h, t, 0)),
        out_shape=jax.ShapeDtypeStruct((H, S, D), jnp.float32),
        compiler_params=pltpu.CompilerParams(
            dimension_semantics=("parallel", "arbitrary")),
    )(bi, q, k, v)
    return out.reshape(B, H, S, D)
